# baseline (speedup 1.0000x reference)
import functools

import jax
import jax.numpy as jnp
from jax import lax
from jax.experimental import pallas as pl
from jax.experimental.pallas import tpu as pltpu
from jax.experimental.pallas import tpu_sc as plsc

_V = 1_000_000
_B = 16384
_D = 64
_NC = 2
_NS = 16
_NW = _NC * _NS
_NBLK = _V // 128
_BASE_BLKS = _NBLK // _NW
_EXTRA = _NBLK % _NW
_TAIL = _NBLK * 128
_CB = 4
_CW = _CB * 128
_OUT_RING = 32
_SENTINEL = 300 << 21

_mesh = plsc.VectorSubcoreMesh(core_axis_name="c", subcore_axis_name="s")


def _i16():
    return lax.iota(jnp.int32, 16)


def _splat(x):
    return jnp.broadcast_to(x, (16,))


@functools.partial(
    pl.kernel,
    mesh=_mesh,
    out_type=jax.ShapeDtypeStruct((_B, _D), jnp.float32),
    scratch_types=[
        pltpu.VMEM((_B + 16,), jnp.int32),
        pltpu.VMEM((_B + 32,), jnp.int32),
        pltpu.VMEM((_D, _CW), jnp.float32),
        pltpu.VMEM((_D, _CW), jnp.float32),
        pltpu.VMEM((_OUT_RING, _D), jnp.float32),
        pltpu.VMEM((_D, _D), jnp.float32),
        pltpu.SemaphoreType.DMA,
        pltpu.SemaphoreType.DMA,
        pltpu.SemaphoreType.DMA,
    ],
    compiler_params=pltpu.CompilerParams(needs_layout_passes=False),
)
def _gather(idx_hbm, tab_hbm, tail_hbm, out_hbm, idx_v, list_k,
            ch_a, ch_b, out_st, tail_v, sem_a, sem_b, sem_o):
    wid = lax.axis_index("s") * _NC + lax.axis_index("c")
    s_w = _BASE_BLKS * wid + jnp.minimum(wid, _EXTRA)
    n_w = _BASE_BLKS + (wid < _EXTRA).astype(jnp.int32)
    is_last = (wid == _NW - 1).astype(jnp.int32)

    pltpu.sync_copy(idx_hbm, idx_v.at[pl.ds(0, _B)])
    pltpu.sync_copy(tail_hbm, tail_v)

    def scan_it(i, cnt):
        ivec = idx_v[pl.ds(i * 16, 16)]
        blk = lax.shift_right_logical(ivec, 7)
        m = (blk >= s_w) & (blk < s_w + n_w + is_last)
        key = (
            lax.shift_left(blk - s_w, 21)
            | lax.shift_left(ivec & 127, 14)
            | (_i16() + i * 16)
        )
        plsc.store_compressed(list_k.at[pl.ds(cnt, 16)], key, mask=m)
        return cnt + plsc.all_reduce_population_count(m)[0]

    cnt = lax.fori_loop(0, _B // 16, scan_it, 0, unroll=False)
    list_k[pl.ds(cnt, 16)] = _splat(jnp.int32(_SENTINEL))
    list_k[pl.ds(cnt + 16, 16)] = _splat(jnp.int32(_SENTINEL))
    nvec = lax.div(cnt + 15, 16)
    nvec2 = lax.div(cnt + 31, 32)

    def emit(key, m_out, src, l):
        slot = lax.rem(m_out, _OUT_RING)
        p = key & 16383

        @pl.when(m_out >= _OUT_RING)
        def _():
            pltpu.make_async_copy(
                out_st.at[pl.ds(0, 1)], out_hbm.at[pl.ds(0, 1)], sem_o
            ).wait()

        for b in range(_D // 16):
            vals = plsc.load_gather(src, [_i16() + 16 * b, _splat(l)])
            plsc.store_scatter(
                out_st, [_splat(slot), _i16() + 16 * b], vals
            )
        pltpu.async_copy(
            out_st.at[pl.ds(slot, 1)], out_hbm.at[pl.ds(p, 1)], sem_o
        )
        return m_out + 1

    def match_chunk(ci, ch, m_out):
        def jt(j, m_out):
            kv0 = list_k[pl.ds(j * 32, 16)]
            kv1 = list_k[pl.ds(j * 32 + 16, 16)]
            mv0 = lax.shift_right_logical(kv0, 23) == ci
            mv1 = lax.shift_right_logical(kv1, 23) == ci
            hit = plsc.all_reduce_population_count(mv0 | mv1)[0]

            def do_lanes(m_in):
                m_cur = m_in
                for kv, mv in ((kv0, mv0), (kv1, mv1)):
                    mvi = mv.astype(jnp.int32)
                    for u in range(16):
                        key = kv[u]
                        l = (
                            lax.shift_right_logical(key, 21) & (_CB - 1)
                        ) * 128 | (lax.shift_right_logical(key, 14) & 127)
                        m_cur = lax.cond(
                            mvi[u] != 0,
                            lambda mc: emit(key, mc, ch, l),
                            lambda mc: mc,
                            m_cur,
                        )
                return m_cur

            return lax.cond(hit != 0, do_lanes, lambda m: m, m_out)

        return lax.fori_loop(0, nvec2, jt, m_out, unroll=False)

    def fire_chunk(ci, sb, sem):
        t = pl.multiple_of((s_w + ci * _CB) * 128, 128)
        pltpu.async_copy(tab_hbm.at[:, pl.ds(t, _CW)], sb, sem)

    def wait_chunk(sb, sem):
        pltpu.make_async_copy(
            tab_hbm.at[:, pl.ds(0, _CW)], sb, sem
        ).wait()

    nch = lax.div(n_w + _CB - 1, _CB)
    fire_chunk(0, ch_a, sem_a)

    def pair(i, m_out):
        ca = 2 * i
        cb = 2 * i + 1

        @pl.when(cb < nch)
        def _():
            fire_chunk(cb, ch_b, sem_b)

        wait_chunk(ch_a, sem_a)
        m_out = match_chunk(ca, ch_a, m_out)

        @pl.when(ca + 2 < nch)
        def _():
            fire_chunk(ca + 2, ch_a, sem_a)

        def do_b(m):
            wait_chunk(ch_b, sem_b)
            return match_chunk(cb, ch_b, m)

        return lax.cond(cb < nch, do_b, lambda m: m, m_out)

    m_out = lax.fori_loop(0, lax.div(nch + 1, 2), pair, 0, unroll=False)

    def tail_jt(j, m_out):
        kv = list_k[pl.ds(j * 16, 16)]
        mv = lax.shift_right_logical(kv, 21) == n_w
        hit = plsc.all_reduce_population_count(mv)[0]

        def do_lanes(m_in):
            mvi = mv.astype(jnp.int32)
            m_cur = m_in
            for u in range(16):
                key = kv[u]
                lt = lax.shift_right_logical(key, 14) & 127
                m_cur = lax.cond(
                    mvi[u] != 0,
                    lambda mc: emit(key, mc, tail_v, lt),
                    lambda mc: mc,
                    m_cur,
                )
            return m_cur

        return lax.cond(hit != 0, do_lanes, lambda m: m, m_out)

    m_out = lax.fori_loop(0, nvec, tail_jt, m_out, unroll=False)

    def drain(i, _):
        @pl.when(i < jnp.minimum(m_out, _OUT_RING))
        def _():
            pltpu.make_async_copy(
                out_st.at[pl.ds(0, 1)], out_hbm.at[pl.ds(0, 1)], sem_o
            ).wait()
        return ()

    lax.fori_loop(0, _OUT_RING, drain, (), unroll=False)


def kernel(idx, latents):
    idx32 = idx.astype(jnp.int32)
    table_t = latents.T
    tail_t = latents[_TAIL:, :].T
    return _gather(idx32, table_t, tail_t)


# speedup vs baseline: 1.4709x; 1.4709x over previous
import functools

import jax
import jax.numpy as jnp
from jax import lax
from jax.experimental import pallas as pl
from jax.experimental.pallas import tpu as pltpu
from jax.experimental.pallas import tpu_sc as plsc

_V = 1_000_000
_B = 16384
_D = 64
_NC = 2
_NS = 16
_NW = _NC * _NS
_B_PER_W = _B // _NW
_PASS = _B_PER_W // 2
_R = 8
_NG = _PASS // 16
_TAIL = (_V // 128) * 128
_TMAX = _TAIL - 128

_mesh = plsc.VectorSubcoreMesh(core_axis_name="c", subcore_axis_name="s")


def _iota16():
    return lax.iota(jnp.int32, 16)


def _splat(x):
    return jnp.broadcast_to(x, (16,))


@functools.partial(
    pl.kernel,
    mesh=_mesh,
    out_type=jax.ShapeDtypeStruct((_B, _D), jnp.float32),
    scratch_types=[
        pltpu.VMEM((_B_PER_W + 32,), jnp.int32),
        pltpu.VMEM((_R, _D, 128), jnp.float32),
        pltpu.VMEM((_PASS, _D), jnp.float32),
        pltpu.VMEM((_D, _D), jnp.float32),
        [pltpu.SemaphoreType.DMA] * _R,
    ],
    compiler_params=pltpu.CompilerParams(needs_layout_passes=False),
)
def _gather(idx_hbm, tab_hbm, tail_hbm, out_hbm, idx_v, sb, rows_v, tail_v,
            sems):
    wid = lax.axis_index("s") * _NC + lax.axis_index("c")
    base = wid * _B_PER_W
    pltpu.sync_copy(
        idx_hbm.at[pl.ds(base, _B_PER_W)], idx_v.at[pl.ds(0, _B_PER_W)]
    )
    pltpu.sync_copy(tail_hbm, tail_v)

    def slab_start(r):
        return pl.multiple_of(
            jnp.minimum(lax.shift_right_logical(r, 7) * 128, _TMAX), 128
        )

    def fire(r, slot):
        pltpu.async_copy(
            tab_hbm.at[:, pl.ds(slab_start(r), 128)], sb.at[slot], sems[slot]
        )

    def wait(slot):
        pltpu.make_async_copy(
            tab_hbm.at[:, pl.ds(0, 128)], sb.at[slot], sems[slot]
        ).wait()

    def extract(r, slot, row):
        l = jnp.minimum(r - slab_start(r), 127)
        for b in range(_D // 16):
            vals = plsc.load_gather(
                sb, [_splat(slot), _iota16() + 16 * b, _splat(l)]
            )
            plsc.store_scatter(
                rows_v, [_splat(row), _iota16() + 16 * b], vals
            )

        @pl.when(r >= _TAIL)
        def _():
            lt = r - _TAIL
            for b in range(_D // 16):
                vals = plsc.load_gather(
                    tail_v, [_iota16() + 16 * b, _splat(lt)]
                )
                plsc.store_scatter(
                    rows_v, [_splat(row), _iota16() + 16 * b], vals
                )

    for h in range(2):
        pbase = h * _PASS
        ivec0 = idx_v[pl.ds(pbase, 16)]
        for u in range(_R):
            fire(ivec0[u], u)

        def body(g, _):
            goff = pbase + g * 16
            ivec = idx_v[pl.ds(goff, 16)]
            ivecn = idx_v[pl.ds(goff + 16, 16)]
            for u in range(16):
                slot = u % _R
                wait(slot)
                extract(ivec[u], slot, g * 16 + u)
                if u < _R:
                    fire(ivec[u + _R], slot)
                else:
                    @pl.when(g < _NG - 1)
                    def _():
                        fire(ivecn[u - _R], slot)
            return ()

        lax.fori_loop(0, _NG, body, (), unroll=False)
        pltpu.sync_copy(rows_v, out_hbm.at[pl.ds(base + pbase, _PASS)])


def kernel(idx, latents):
    idx32 = idx.astype(jnp.int32)
    table_t = latents.T
    tail_t = latents[_TAIL:, :].T
    return _gather(idx32, table_t, tail_t)


# speedup vs baseline: 1.9922x; 1.3544x over previous
import functools

import jax
import jax.numpy as jnp
from jax import lax
from jax.experimental import pallas as pl
from jax.experimental.pallas import tpu as pltpu
from jax.experimental.pallas import tpu_sc as plsc

_V = 1_000_000
_B = 16384
_D = 64
_NC = 2
_NS = 16
_NW = _NC * _NS
_NBLK = _V // 128
_BASE_BLKS = _NBLK // _NW
_EXTRA = _NBLK % _NW
_TAIL = _NBLK * 128
_CB = 4
_CW = _CB * 128
_NREG = 8
_OUT_RING = 32
_SENTINEL = 300 << 21

_mesh = plsc.VectorSubcoreMesh(core_axis_name="c", subcore_axis_name="s")


def _i16():
    return lax.iota(jnp.int32, 16)


def _splat(x):
    return jnp.broadcast_to(x, (16,))


_lane0 = None


@functools.partial(
    pl.kernel,
    mesh=_mesh,
    out_type=jax.ShapeDtypeStruct((_B, _D), jnp.float32),
    scratch_types=[
        pltpu.VMEM((_B + 16,), jnp.int32),
        pltpu.VMEM((_B + 32,), jnp.int32),
        pltpu.VMEM((_B + 32,), jnp.int32),
        pltpu.VMEM((16,), jnp.int32),
        pltpu.VMEM((_D, _CW), jnp.float32),
        pltpu.VMEM((_D, _CW), jnp.float32),
        pltpu.VMEM((_OUT_RING, _D), jnp.float32),
        pltpu.VMEM((_D, _D), jnp.float32),
        pltpu.SemaphoreType.DMA,
        pltpu.SemaphoreType.DMA,
        pltpu.SemaphoreType.DMA,
    ],
    compiler_params=pltpu.CompilerParams(needs_layout_passes=False),
)
def _gather(idx_hbm, tab_hbm, tail_hbm, out_hbm, idx_v, list_k, list2,
            offs_v, ch_a, ch_b, out_st, tail_v, sem_a, sem_b, sem_o):
    wid = lax.axis_index("s") * _NC + lax.axis_index("c")
    s_w = _BASE_BLKS * wid + jnp.minimum(wid, _EXTRA)
    n_w = _BASE_BLKS + (wid < _EXTRA).astype(jnp.int32)
    is_last = (wid == _NW - 1).astype(jnp.int32)
    lane0 = _i16() == 0

    pltpu.sync_copy(idx_hbm, idx_v.at[pl.ds(0, _B)])
    pltpu.sync_copy(tail_hbm, tail_v)

    def scan_it(i, cnt):
        ivec = idx_v[pl.ds(i * 16, 16)]
        blk = lax.shift_right_logical(ivec, 7)
        m = (blk >= s_w) & (blk < s_w + n_w + is_last)
        key = (
            lax.shift_left(blk - s_w, 21)
            | lax.shift_left(ivec & 127, 14)
            | (_i16() + i * 16)
        )
        plsc.store_compressed(list_k.at[pl.ds(cnt, 16)], key, mask=m)
        return cnt + plsc.all_reduce_population_count(m)[0]

    cnt = lax.fori_loop(0, _B // 16, scan_it, 0, unroll=False)
    list_k[pl.ds(cnt, 16)] = _splat(jnp.int32(_SENTINEL))
    nvec = lax.div(cnt + 15, 16)

    cnt2 = 0
    for r in range(_NREG):
        plsc.store_scatter(
            offs_v, [_splat(r)], _splat(cnt2), mask=lane0
        )

        def rb(j, c, r=r):
            kv = list_k[pl.ds(j * 16, 16)]
            mv = lax.shift_right_logical(kv, 26) == r
            plsc.store_compressed(list2.at[pl.ds(c, 16)], kv, mask=mv)
            return c + plsc.all_reduce_population_count(mv)[0]

        cnt2 = lax.fori_loop(0, nvec, rb, cnt2, unroll=False)
    for r in range(_NREG, 16):
        plsc.store_scatter(offs_v, [_splat(r)], _splat(cnt2), mask=lane0)
    list2[pl.ds(cnt2, 16)] = _splat(jnp.int32(_SENTINEL))
    list2[pl.ds(cnt2 + 16, 16)] = _splat(jnp.int32(_SENTINEL))

    def emit(key, m_out, src, l):
        slot = lax.rem(m_out, _OUT_RING)
        p = key & 16383

        @pl.when(m_out >= _OUT_RING)
        def _():
            pltpu.make_async_copy(
                out_st.at[pl.ds(0, 1)], out_hbm.at[pl.ds(0, 1)], sem_o
            ).wait()

        for b in range(_D // 16):
            vals = plsc.load_gather(src, [_i16() + 16 * b, _splat(l)])
            plsc.store_scatter(
                out_st, [_splat(slot), _i16() + 16 * b], vals
            )
        pltpu.async_copy(
            out_st.at[pl.ds(slot, 1)], out_hbm.at[pl.ds(p, 1)], sem_o
        )
        return m_out + 1

    def match_chunk(ci, ch, m_out):
        reg = lax.shift_right_logical(ci, 3)
        o0 = plsc.load_gather(offs_v, [_splat(reg)])[0]
        o1 = plsc.load_gather(offs_v, [_splat(reg + 1)])[0]
        jlo = lax.shift_right_logical(o0, 4)
        jhi = lax.shift_right_logical(o1 + 15, 4)

        def jt(j, m_out):
            kv = list2[pl.ds(j * 16, 16)]
            mv = lax.shift_right_logical(kv, 23) == ci
            hit = plsc.all_reduce_population_count(mv)[0]

            def do_lanes(m_in):
                mvi = mv.astype(jnp.int32)
                m_cur = m_in
                for u in range(16):
                    key = kv[u]
                    l = (
                        lax.shift_right_logical(key, 21) & (_CB - 1)
                    ) * 128 | (lax.shift_right_logical(key, 14) & 127)
                    m_cur = lax.cond(
                        mvi[u] != 0,
                        lambda mc: emit(key, mc, ch, l),
                        lambda mc: mc,
                        m_cur,
                    )
                return m_cur

            return lax.cond(hit != 0, do_lanes, lambda m: m, m_out)

        return lax.fori_loop(jlo, jhi, jt, m_out, unroll=False)

    def fire_chunk(ci, sb, sem):
        t = pl.multiple_of((s_w + ci * _CB) * 128, 128)
        pltpu.async_copy(tab_hbm.at[:, pl.ds(t, _CW)], sb, sem)

    def wait_chunk(sb, sem):
        pltpu.make_async_copy(
            tab_hbm.at[:, pl.ds(0, _CW)], sb, sem
        ).wait()

    nch = lax.div(n_w + _CB - 1, _CB)
    fire_chunk(0, ch_a, sem_a)

    def pair(i, m_out):
        ca = 2 * i
        cb = 2 * i + 1

        @pl.when(cb < nch)
        def _():
            fire_chunk(cb, ch_b, sem_b)

        wait_chunk(ch_a, sem_a)
        m_out = match_chunk(ca, ch_a, m_out)

        @pl.when(ca + 2 < nch)
        def _():
            fire_chunk(ca + 2, ch_a, sem_a)

        def do_b(m):
            wait_chunk(ch_b, sem_b)
            return match_chunk(cb, ch_b, m)

        return lax.cond(cb < nch, do_b, lambda m: m, m_out)

    m_out = lax.fori_loop(0, lax.div(nch + 1, 2), pair, 0, unroll=False)

    def tail_jt(j, m_out):
        kv = list_k[pl.ds(j * 16, 16)]
        mv = lax.shift_right_logical(kv, 21) == n_w
        hit = plsc.all_reduce_population_count(mv)[0]

        def do_lanes(m_in):
            mvi = mv.astype(jnp.int32)
            m_cur = m_in
            for u in range(16):
                key = kv[u]
                lt = lax.shift_right_logical(key, 14) & 127
                m_cur = lax.cond(
                    mvi[u] != 0,
                    lambda mc: emit(key, mc, tail_v, lt),
                    lambda mc: mc,
                    m_cur,
                )
            return m_cur

        return lax.cond(hit != 0, do_lanes, lambda m: m, m_out)

    m_out = lax.fori_loop(0, nvec, tail_jt, m_out, unroll=False)

    def drain(i, _):
        @pl.when(i < jnp.minimum(m_out, _OUT_RING))
        def _():
            pltpu.make_async_copy(
                out_st.at[pl.ds(0, 1)], out_hbm.at[pl.ds(0, 1)], sem_o
            ).wait()
        return ()

    lax.fori_loop(0, _OUT_RING, drain, (), unroll=False)


def kernel(idx, latents):
    idx32 = idx.astype(jnp.int32)
    table_t = latents.T
    tail_t = latents[_TAIL:, :].T
    return _gather(idx32, table_t, tail_t)
